# baseline (speedup 1.0000x reference)
import functools

import jax
import jax.numpy as jnp
from jax import lax
from jax.experimental import pallas as pl
from jax.experimental.pallas import tpu as pltpu
from jax.experimental.pallas import tpu_sc as plsc

NC = 2
NS = 16
NW = NC * NS
B = 16384
D = 64
ROWS_PER_W = B // NW
CHUNK = 128
NCHUNK = ROWS_PER_W // CHUNK
INV_B = 1.0 / B


def _body(labels_hbm, features_hbm, proxy_hbm, out_hbm,
          lab_v, gidx_v, feat_v0, feat_v1, rows_v0, rows_v1, acc_v,
          gsem0, gsem1, fsem0, fsem1):
    wid = lax.axis_index("s") * NC + lax.axis_index("c")
    base = wid * ROWS_PER_W

    pltpu.sync_copy(labels_hbm.at[pl.ds(wid * NCHUNK, NCHUNK)], lab_v)

    for j in range(NCHUNK):
        for g in range(8):
            l_v = lab_v[j, pl.ds(g * 16, 16)]
            gidx_v[j, pl.ds(g * 16, 16)] = lax.shift_right_logical(l_v, 1)

    feat_bufs = [feat_v0, feat_v1]
    rows_bufs = [rows_v0, rows_v1]
    gsems = [gsem0, gsem1]
    fsems = [fsem0, fsem1]

    def start(j):
        g = pltpu.async_copy(proxy_hbm.at[gidx_v.at[j]], rows_bufs[j % 2],
                             gsems[j % 2])
        f = pltpu.async_copy(
            features_hbm.at[:, pl.ds(base + j * CHUNK, CHUNK)],
            feat_bufs[j % 2], fsems[j % 2])
        return g, f

    iota = lax.iota(jnp.int32, 16)
    accs = [jnp.zeros((16,), jnp.float32) for _ in range(4)]
    pend = start(0)
    for j in range(NCHUNK):
        nxt = start(j + 1) if j + 1 < NCHUNK else None
        pend[0].wait()
        pend[1].wait()
        rb = rows_bufs[j % 2]
        fb = feat_bufs[j % 2]

        def group(g, carry):
            l_v = lab_v[j, pl.ds(g * 16, 16)]
            row_v = g * 16 + iota
            col_v = lax.shift_left(jnp.bitwise_and(l_v, 1), 6)
            outs = list(carry)
            for c in range(D):
                pv = plsc.load_gather(rb, [row_v, col_v + c])
                fv = fb[c, pl.ds(g * 16, 16)]
                d = jnp.abs(fv - pv)
                m = jnp.minimum(d, 1.0)
                outs[c % 4] = outs[c % 4] + m * (d - 0.5 * m)
            return tuple(outs)

        accs = list(lax.fori_loop(0, 8, group, tuple(accs)))
        pend = nxt

    zeros = jnp.zeros((16,), jnp.float32)
    for k in range(4):
        acc_v[pl.ds(k * 16, 16)] = accs[k] * INV_B
    for k in range(4, 8):
        acc_v[pl.ds(k * 16, 16)] = zeros
    pltpu.sync_copy(acc_v, out_hbm.at[wid])


@jax.jit
def kernel(features, proxy, labels):
    labels2d = labels.astype(jnp.int32).reshape(B // CHUNK, CHUNK)
    features_t = features.T
    proxy128 = proxy.reshape(50000, 128)
    run = pl.kernel(
        _body,
        out_type=jax.ShapeDtypeStruct((NW, 128), jnp.float32),
        mesh=plsc.VectorSubcoreMesh(core_axis_name="c", subcore_axis_name="s"),
        compiler_params=pltpu.CompilerParams(
            use_tc_tiling_on_sc=True, needs_layout_passes=False),
        scratch_types=[
            pltpu.VMEM((NCHUNK, CHUNK), jnp.int32),
            pltpu.VMEM((NCHUNK, CHUNK), jnp.int32),
            pltpu.VMEM((D, CHUNK), jnp.float32),
            pltpu.VMEM((D, CHUNK), jnp.float32),
            pltpu.VMEM((CHUNK, 128), jnp.float32),
            pltpu.VMEM((CHUNK, 128), jnp.float32),
            pltpu.VMEM((128,), jnp.float32),
            pltpu.SemaphoreType.DMA,
            pltpu.SemaphoreType.DMA,
            pltpu.SemaphoreType.DMA,
            pltpu.SemaphoreType.DMA,
        ],
    )
    partials = run(labels2d, features_t, proxy128)
    return jnp.sum(partials)
